# baseline (device time: 33025 ns/iter reference)
import jax
import jax.numpy as jnp
from jax import lax
from jax.experimental import pallas as pl
from jax.experimental.pallas import tpu as pltpu

N_DEV = 4
SQ = 512
D = 1024
DH = 128
HQ_LOCAL = 8
GROUP = 4
CHUNK = SQ // N_DEV
SCALE = 0.08838834764831843

_ORDER = (2, 1, 3)


def kernel(x, Wq, Wo, Wk, Wv):
    my = lax.axis_index("i")
    Wk_loc = lax.dynamic_slice_in_dim(Wk, my * 2 * DH, 2 * DH, axis=1)
    Wv_loc = lax.dynamic_slice_in_dim(Wv, my * 2 * DH, 2 * DH, axis=1)
    x2 = x.reshape(SQ, D)

    def body(x_ref, wq_ref, wo_ref, wk_ref, wv_ref, out_ref,
             send_ref, rs_recv_ref, ag_ref,
             rs_send_sems, rs_recv_sems, ag_send_sems, ag_recv_sems):
        p = lax.axis_index("i")

        barrier_sem = pltpu.get_barrier_semaphore()
        for k in range(1, N_DEV):
            pl.semaphore_signal(
                barrier_sem, inc=1,
                device_id=(lax.rem(p + k, N_DEV),),
                device_id_type=pl.DeviceIdType.MESH,
            )
        pl.semaphore_wait(barrier_sem, N_DEV - 1)

        xv = x_ref[:, :]
        K = jnp.dot(xv, wk_ref[:, :], preferred_element_type=jnp.float32)
        V = jnp.dot(xv, wv_ref[:, :], preferred_element_type=jnp.float32)

        def compute_chunk(t):
            xc = x_ref[pl.ds(t * CHUNK, CHUNK), :]
            Qc = jnp.dot(xc, wq_ref[:, :], preferred_element_type=jnp.float32)
            outs = []
            for j in range(HQ_LOCAL):
                qj = Qc[:, j * DH:(j + 1) * DH]
                g = j // GROUP
                kg = K[:, g * DH:(g + 1) * DH]
                vg = V[:, g * DH:(g + 1) * DH]
                s = lax.dot_general(
                    qj, kg, (((1,), (1,)), ((), ())),
                    preferred_element_type=jnp.float32,
                ) * SCALE
                m = jnp.max(s, axis=1, keepdims=True)
                pj = jnp.exp(s - m)
                l = jnp.sum(pj, axis=1, keepdims=True)
                outs.append(
                    jnp.dot(pj, vg, preferred_element_type=jnp.float32) / l)
            attn = jnp.concatenate(outs, axis=1)
            return jnp.dot(attn, wo_ref[:, :],
                           preferred_element_type=jnp.float32)

        rs = []
        for k in _ORDER:
            t = lax.rem(p + k, N_DEV)
            send_ref[t] = compute_chunk(t).astype(jnp.bfloat16)
            r = pltpu.make_async_remote_copy(
                src_ref=send_ref.at[t],
                dst_ref=rs_recv_ref.at[N_DEV - 1 - k],
                send_sem=rs_send_sems.at[k - 1],
                recv_sem=rs_recv_sems.at[N_DEV - 1 - k],
                device_id=(t,),
                device_id_type=pl.DeviceIdType.MESH,
            )
            r.start()
            rs.append(r)

        acc = compute_chunk(p)
        for i, k in enumerate(_ORDER):
            rs[i].wait_recv()
            acc = acc + rs_recv_ref[N_DEV - 1 - k].astype(jnp.float32)
        out_ref[p] = acc
        ag_ref[p] = acc.astype(jnp.bfloat16)

        ag = {}
        for k in _ORDER:
            t = lax.rem(p + k, N_DEV)
            a = pltpu.make_async_remote_copy(
                src_ref=ag_ref.at[p],
                dst_ref=ag_ref.at[p],
                send_sem=ag_send_sems.at[k - 1],
                recv_sem=ag_recv_sems.at[N_DEV - 1 - k],
                device_id=(t,),
                device_id_type=pl.DeviceIdType.MESH,
            )
            a.start()
            ag[k] = a
        for r in rs:
            r.wait_send()
        for k in (1, 3, 2):
            ag[k].wait_recv()
            s = lax.rem(p + N_DEV - k, N_DEV)
            out_ref[s] = ag_ref[s].astype(jnp.float32)
        for k in _ORDER:
            ag[k].wait_send()

    out = pl.pallas_call(
        body,
        out_shape=jax.ShapeDtypeStruct((N_DEV, CHUNK, D), jnp.float32),
        in_specs=[pl.BlockSpec(memory_space=pltpu.VMEM)] * 5,
        out_specs=pl.BlockSpec(memory_space=pltpu.VMEM),
        scratch_shapes=[
            pltpu.VMEM((N_DEV, CHUNK, D), jnp.bfloat16),
            pltpu.VMEM((N_DEV - 1, CHUNK, D), jnp.bfloat16),
            pltpu.VMEM((N_DEV, CHUNK, D), jnp.bfloat16),
            pltpu.SemaphoreType.DMA((N_DEV - 1,)),
            pltpu.SemaphoreType.DMA((N_DEV - 1,)),
            pltpu.SemaphoreType.DMA((N_DEV - 1,)),
            pltpu.SemaphoreType.DMA((N_DEV - 1,)),
        ],
        compiler_params=pltpu.CompilerParams(collective_id=0),
    )(x2, Wq, Wo, Wk_loc, Wv_loc)
    return out.reshape(1, SQ, D)


# device time: 31068 ns/iter; 1.0630x vs baseline; 1.0630x over previous
import jax
import jax.numpy as jnp
from jax import lax
from jax.experimental import pallas as pl
from jax.experimental.pallas import tpu as pltpu

N_DEV = 4
SQ = 512
D = 1024
DH = 128
HQ_LOCAL = 8
GROUP = 4
CHUNK = SQ // N_DEV
SCALE = 0.08838834764831843

_ORDER = (2, 1, 3)


def kernel(x, Wq, Wo, Wk, Wv):
    my = lax.axis_index("i")
    Wk_loc = lax.dynamic_slice_in_dim(Wk, my * 2 * DH, 2 * DH, axis=1)
    Wv_loc = lax.dynamic_slice_in_dim(Wv, my * 2 * DH, 2 * DH, axis=1)
    bf = jnp.bfloat16
    xb = x.astype(bf)
    Wqb = Wq.astype(bf)
    Wob = Wo.astype(bf)
    Wkb = Wk_loc.astype(bf)
    Wvb = Wv_loc.astype(bf)

    def body(x_ref, wq_ref, wo_ref, wk_ref, wv_ref, out_ref,
             attn_ref, send_ref, rs_recv_ref, ag_ref,
             rs_send_sems, rs_recv_sems, ag_send_sems, ag_recv_sems):
        p = lax.axis_index("i")

        xv = x_ref[0]
        Q = jnp.dot(xv, wq_ref[:, :], preferred_element_type=jnp.float32)
        K = jnp.dot(xv, wk_ref[:, :], preferred_element_type=jnp.float32)
        V = jnp.dot(xv, wv_ref[:, :], preferred_element_type=jnp.float32)
        Qb = Q.astype(bf)
        Kb = K.astype(bf)
        Vb = V.astype(bf)
        outs = []
        for j in range(HQ_LOCAL):
            qj = Qb[:, j * DH:(j + 1) * DH]
            g = j // GROUP
            kg = Kb[:, g * DH:(g + 1) * DH]
            vg = Vb[:, g * DH:(g + 1) * DH]
            s = lax.dot_general(
                qj, kg, (((1,), (1,)), ((), ())),
                preferred_element_type=jnp.float32,
            ) * SCALE
            m = jnp.max(s, axis=1, keepdims=True)
            pj = jnp.exp(s - m)
            l = jnp.sum(pj, axis=1, keepdims=True)
            o = jnp.dot(pj.astype(bf), vg, preferred_element_type=jnp.float32)
            outs.append(o / l)
        attn = jnp.concatenate(outs, axis=1)
        attn_ref[...] = attn.astype(bf).reshape(N_DEV, CHUNK, D)

        barrier_sem = pltpu.get_barrier_semaphore()
        for k in range(1, N_DEV):
            pl.semaphore_signal(
                barrier_sem, inc=1,
                device_id=(lax.rem(p + k, N_DEV),),
                device_id_type=pl.DeviceIdType.MESH,
            )
        pl.semaphore_wait(barrier_sem, N_DEV - 1)

        def wo_chunk(t):
            return jnp.dot(attn_ref[t], wo_ref[:, :],
                           preferred_element_type=jnp.float32)

        rs = []
        for k in _ORDER:
            t = lax.rem(p + k, N_DEV)
            send_ref[t] = wo_chunk(t).astype(bf)
            r = pltpu.make_async_remote_copy(
                src_ref=send_ref.at[t],
                dst_ref=rs_recv_ref.at[N_DEV - 1 - k],
                send_sem=rs_send_sems.at[k - 1],
                recv_sem=rs_recv_sems.at[N_DEV - 1 - k],
                device_id=(t,),
                device_id_type=pl.DeviceIdType.MESH,
            )
            r.start()
            rs.append(r)

        acc = wo_chunk(p)
        for i, k in enumerate(_ORDER):
            rs[i].wait_recv()
            acc = acc + rs_recv_ref[N_DEV - 1 - k].astype(jnp.float32)
        out_ref[0, pl.ds(p * CHUNK, CHUNK), :] = acc
        ag_ref[p] = acc.astype(bf)

        ag = {}
        for k in _ORDER:
            t = lax.rem(p + k, N_DEV)
            a = pltpu.make_async_remote_copy(
                src_ref=ag_ref.at[p],
                dst_ref=ag_ref.at[p],
                send_sem=ag_send_sems.at[k - 1],
                recv_sem=ag_recv_sems.at[N_DEV - 1 - k],
                device_id=(t,),
                device_id_type=pl.DeviceIdType.MESH,
            )
            a.start()
            ag[k] = a
        for r in rs:
            r.wait_send()
        for k in (1, 3, 2):
            ag[k].wait_recv()
            s = lax.rem(p + N_DEV - k, N_DEV)
            out_ref[0, pl.ds(s * CHUNK, CHUNK), :] = ag_ref[s].astype(jnp.float32)
        for k in _ORDER:
            ag[k].wait_send()

    return pl.pallas_call(
        body,
        out_shape=jax.ShapeDtypeStruct((1, SQ, D), jnp.float32),
        in_specs=[pl.BlockSpec(memory_space=pltpu.VMEM)] * 5,
        out_specs=pl.BlockSpec(memory_space=pltpu.VMEM),
        scratch_shapes=[
            pltpu.VMEM((N_DEV, CHUNK, D), jnp.bfloat16),
            pltpu.VMEM((N_DEV, CHUNK, D), jnp.bfloat16),
            pltpu.VMEM((N_DEV - 1, CHUNK, D), jnp.bfloat16),
            pltpu.VMEM((N_DEV, CHUNK, D), jnp.bfloat16),
            pltpu.SemaphoreType.DMA((N_DEV - 1,)),
            pltpu.SemaphoreType.DMA((N_DEV - 1,)),
            pltpu.SemaphoreType.DMA((N_DEV - 1,)),
            pltpu.SemaphoreType.DMA((N_DEV - 1,)),
        ],
        compiler_params=pltpu.CompilerParams(collective_id=0),
    )(xb, Wqb, Wob, Wkb, Wvb)
